# Initial kernel scaffold; baseline (speedup 1.0000x reference)
import jax
import jax.numpy as jnp
from jax.experimental import pallas as pl


def kernel(edge_coulomb, edge_lengths, node_from, node_to, node_graph_index, w1, b1, ln1_g, ln1_b, w2, b2, ln2_g, ln2_b, ow1, ob1, ow2, ob2):
    raise NotImplementedError("write your pallas kernel here")


# speedup vs baseline: 63.8857x; 63.8857x over previous
import functools

import jax
import jax.numpy as jnp
from jax import lax
from jax.experimental import pallas as pl
from jax.experimental.pallas import tpu as pltpu
from jax.experimental.pallas import tpu_sc as plsc

N_NODES = 10000
N_EDGES = 160000
N_GRAPHS = 100
D = 32
H1 = 128
OUT_DIM = 4
ROUNDS = 5
EPS = 1e-10
SLOPE = 0.01

NC = 2
NS = 16
NW = NC * NS
EPW = N_EDGES // NW
CH = 1000
NCH = EPW // CH
RPW = 1000
NTI = N_NODES // RPW

_f32 = functools.partial(jnp.asarray, dtype=jnp.float32)


@functools.cache
def _sc_kernels():
    mesh = plsc.VectorSubcoreMesh(core_axis_name="c", subcore_axis_name="s",
                                  num_cores=NC, num_subcores=NS)

    @functools.partial(
        pl.kernel,
        out_type=(jax.ShapeDtypeStruct((N_EDGES, D), jnp.float32),
                  jax.ShapeDtypeStruct((N_EDGES, D), jnp.float32)),
        mesh=mesh,
        compiler_params=pltpu.CompilerParams(use_tc_tiling_on_sc=False),
        scratch_types=[pltpu.VMEM((CH,), jnp.int32),
                       pltpu.VMEM((CH, D), jnp.float32),
                       pltpu.VMEM((CH, D), jnp.float32),
                       pltpu.SemaphoreType.DMA],
    )
    def sc_gather(p, idx_hbm, g0, g1, idx_v, r0, r1, sem):
        c = lax.axis_index("c")
        s = lax.axis_index("s")
        base = (s * jnp.int32(NC) + c) * jnp.int32(EPW)

        def body(i, carry):
            off = base + i * jnp.int32(CH)
            pltpu.sync_copy(idx_hbm.at[pl.ds(off, CH)], idx_v)
            cp0 = pltpu.async_copy(p.at[jnp.int32(0)].at[idx_v], r0, sem)
            cp1 = pltpu.async_copy(p.at[jnp.int32(1)].at[idx_v], r1, sem)
            cp0.wait()
            cp1.wait()
            pltpu.sync_copy(r0, g0.at[pl.ds(off, CH)])
            pltpu.sync_copy(r1, g1.at[pl.ds(off, CH)])
            return carry

        lax.fori_loop(jnp.int32(0), jnp.int32(NCH), body, jnp.int32(0))

    @functools.partial(
        pl.kernel,
        out_type=jax.ShapeDtypeStruct((NC, N_NODES, D), jnp.float32),
        mesh=mesh,
        compiler_params=pltpu.CompilerParams(use_tc_tiling_on_sc=False),
        scratch_types=[pltpu.VMEM((CH,), jnp.int32),
                       pltpu.VMEM((CH, D), jnp.float32),
                       pltpu.VMEM_SHARED((N_NODES, D), jnp.float32)],
    )
    def sc_scatter(pp, msg, idx_hbm, q, idx_v, rows_v, acc):
        c = lax.axis_index("c")
        s = lax.axis_index("s")
        rs = s * jnp.int32(RPW)

        @pl.when(s < NTI)
        def _():
            pltpu.sync_copy(pp.at[c].at[pl.ds(rs, RPW)], rows_v)
            pltpu.sync_copy(rows_v, acc.at[pl.ds(rs, RPW)])

        plsc.subcore_barrier()

        base = (s * jnp.int32(NC) + c) * jnp.int32(EPW)

        def body(i, carry):
            off = base + i * jnp.int32(CH)
            pltpu.sync_copy(idx_hbm.at[pl.ds(off, CH)], idx_v)
            pltpu.sync_copy(msg.at[pl.ds(off, CH)], rows_v)
            pltpu.sync_copy(rows_v, acc.at[idx_v], add=True)
            return carry

        lax.fori_loop(jnp.int32(0), jnp.int32(NCH), body, jnp.int32(0))
        plsc.subcore_barrier()

        @pl.when(s < NTI)
        def _():
            pltpu.sync_copy(acc.at[pl.ds(rs, RPW)], rows_v)
            pltpu.sync_copy(rows_v, q.at[c].at[pl.ds(rs, RPW)])

    return sc_gather, sc_scatter


BE = 2000


def _ln(x, g, b):
    m = jnp.mean(x, axis=-1, keepdims=True)
    v = jnp.mean((x - m) ** 2, axis=-1, keepdims=True)
    return (x - m) / jnp.sqrt(v + 1e-5) * g + b


def _leaky(x):
    return jnp.where(x >= 0, x, SLOPE * x)


def _mlp_body(first, *refs):
    if first:
        (ec, w1b, b1, g1, bb1, w2, b2, g2, bb2, out) = refs
        h = ec[...] * w1b[...] + b1[...]
    else:
        (ga, gb, ec, w1a, w1b, b1, g1, bb1, w2, b2, g2, bb2, out) = refs
        x = ga[...] + gb[...]
        h = (jnp.dot(x, w1a[...], preferred_element_type=jnp.float32,
                     precision=lax.Precision.HIGHEST)
             + ec[...] * w1b[...] + b1[...])
    h = _leaky(_ln(h, g1[...], bb1[...]))
    u = jnp.dot(h, w2[...], preferred_element_type=jnp.float32,
                     precision=lax.Precision.HIGHEST) + b2[...]
    out[...] = _leaky(_ln(u, g2[...], bb2[...]))


def _make_mlp(first):
    full = lambda i: (jnp.int32(0), jnp.int32(0))
    edge = lambda i: (i, jnp.int32(0))
    in_specs = []
    if not first:
        in_specs += [pl.BlockSpec((BE, D), edge)] * 2
    in_specs += [pl.BlockSpec((BE, 1), edge)]
    if not first:
        in_specs += [pl.BlockSpec((D, H1), full)]
    in_specs += [pl.BlockSpec((1, H1), full)] * 4
    in_specs += [pl.BlockSpec((H1, D), full)]
    in_specs += [pl.BlockSpec((1, D), full)] * 3
    return pl.pallas_call(
        functools.partial(_mlp_body, first),
        grid=(N_EDGES // BE,),
        in_specs=in_specs,
        out_specs=pl.BlockSpec((BE, D), edge),
        out_shape=jax.ShapeDtypeStruct((N_EDGES, D), jnp.float32),
    )


_mlp_first = _make_mlp(True)
_mlp = _make_mlp(False)


NB = 2000
NGB = N_NODES // NB


def _pool_body(ngi, p0, p1, ow1, ob1, ow2, ob2, out, acc):
    i = pl.program_id(0)

    @pl.when(i == 0)
    def _():
        acc[...] = jnp.zeros_like(acc)

    rows = p0[...] + p1[...]
    gids = ngi[0]
    giota = lax.broadcasted_iota(jnp.int32, (N_GRAPHS, NB), 0)
    oh = (giota == gids).astype(jnp.float32)
    acc[...] += jnp.dot(oh, rows, preferred_element_type=jnp.float32,
                     precision=lax.Precision.HIGHEST)

    @pl.when(i == NGB - 1)
    def _():
        ev = jnp.dot(acc[...], ow1[...],
                     preferred_element_type=jnp.float32,
                     precision=lax.Precision.HIGHEST) + ob1[...]
        ev = jnp.dot(ev, ow2[...],
                     preferred_element_type=jnp.float32,
                     precision=lax.Precision.HIGHEST) + ob2[...]
        sp = jnp.maximum(ev, 0.0) + jnp.log1p(jnp.exp(-jnp.abs(ev)))
        col = lax.broadcasted_iota(jnp.int32, (N_GRAPHS, OUT_DIM), 1)
        out[...] = jnp.where(col == 0, ev,
                             sp + EPS + (col == 2).astype(jnp.float32))


_pool_head = pl.pallas_call(
    _pool_body,
    grid=(NGB,),
    in_specs=[pl.BlockSpec((1, 1, NB), lambda i: (i, jnp.int32(0), jnp.int32(0))),
              pl.BlockSpec((NB, D), lambda i: (i, jnp.int32(0))),
              pl.BlockSpec((NB, D), lambda i: (i, jnp.int32(0))),
              pl.BlockSpec((D, H1), lambda i: (jnp.int32(0), jnp.int32(0))),
              pl.BlockSpec((1, H1), lambda i: (jnp.int32(0), jnp.int32(0))),
              pl.BlockSpec((H1, OUT_DIM), lambda i: (jnp.int32(0), jnp.int32(0))),
              pl.BlockSpec((1, OUT_DIM), lambda i: (jnp.int32(0), jnp.int32(0)))],
    out_specs=pl.BlockSpec((N_GRAPHS, OUT_DIM), lambda i: (jnp.int32(0), jnp.int32(0))),
    out_shape=jax.ShapeDtypeStruct((N_GRAPHS, OUT_DIM), jnp.float32),
    scratch_shapes=[pltpu.VMEM((N_GRAPHS, D), jnp.float32)],
)


def kernel(edge_coulomb, edge_lengths, node_from, node_to, node_graph_index,
           w1, b1, ln1_g, ln1_b, w2, b2, ln2_g, ln2_b, ow1, ob1, ow2, ob2):
    ec = _f32(edge_coulomb)
    nf = node_from.astype(jnp.int32)
    nt = node_to.astype(jnp.int32)
    ngi = node_graph_index.astype(jnp.int32).reshape(NGB, 1, NB)

    w1a = _f32(w1[:D])
    w1b = _f32(w1[D:])
    b1f = _f32(b1).reshape(1, H1)
    g1f = _f32(ln1_g).reshape(1, H1)
    bb1f = _f32(ln1_b).reshape(1, H1)
    w2f = _f32(w2)
    b2f = _f32(b2).reshape(1, D)
    g2f = _f32(ln2_g).reshape(1, D)
    bb2f = _f32(ln2_b).reshape(1, D)
    ow1f = _f32(ow1)
    ob1f = _f32(ob1).reshape(1, H1)
    ow2f = _f32(ow2)
    ob2f = _f32(ob2).reshape(1, OUT_DIM)

    zeros = jnp.zeros((NC, N_NODES, D), jnp.float32)
    _sc_gather, _sc_scatter = _sc_kernels()

    msg = _mlp_first(ec, w1b, b1f, g1f, bb1f, w2f, b2f, g2f, bb2f)
    p = _sc_scatter(zeros, msg, nt)
    for _ in range(ROUNDS - 1):
        g0, g1 = _sc_gather(p, nf)
        msg = _mlp(g0, g1, ec, w1a, w1b, b1f, g1f, bb1f,
                   w2f, b2f, g2f, bb2f)
        p = _sc_scatter(p, msg, nt)

    out = _pool_head(ngi, p[0], p[1], ow1f, ob1f, ow2f, ob2f)
    return out.astype(jnp.float64)
